# baseline (speedup 1.0000x reference)
import functools

import jax
import jax.numpy as jnp
from jax import lax
from jax.experimental import pallas as pl
from jax.experimental.pallas import tpu as pltpu
from jax.experimental.pallas import tpu_sc as plsc

_N_FIELDS = 26
_VOCAB = 1000
_EMB = 128
_N_CONT = 13
_BATCH = 16384

_NW = 32


def _sc_gather(flat_table, gidx, chunk, n_chunks):
    mesh = plsc.VectorSubcoreMesh(core_axis_name="c", subcore_axis_name="s")
    rows_per_w = n_chunks * chunk
    _CHUNK = chunk

    @functools.partial(
        pl.kernel,
        mesh=mesh,
        out_type=jax.ShapeDtypeStruct((_NW * rows_per_w, _EMB), jnp.float32),
        scratch_types=[
            pltpu.VMEM((n_chunks, chunk), jnp.int32),
            [pltpu.VMEM((chunk, _EMB), jnp.float32)] * 4,
            [pltpu.SemaphoreType.DMA] * 4,
            [pltpu.SemaphoreType.DMA] * 4,
        ],
    )
    def k(tbl_hbm, idx_hbm, out_hbm, idx_v, bufs, gsems, wsems):
        wid = lax.axis_index("s") * 2 + lax.axis_index("c")
        base = wid * rows_per_w
        pltpu.sync_copy(idx_hbm.at[wid], idx_v)

        def start_gather(j, b):
            pltpu.async_copy(tbl_hbm.at[idx_v.at[j]], bufs[b], gsems[b])

        def wait_gather(j, b):
            pltpu.make_async_copy(tbl_hbm.at[idx_v.at[j]], bufs[b],
                                  gsems[b]).wait()

        def start_wb(j, b):
            pltpu.async_copy(bufs[b],
                             out_hbm.at[pl.ds(base + j * _CHUNK, _CHUNK)],
                             wsems[b])

        def wait_wb(j, b):
            pltpu.make_async_copy(bufs[b],
                                  out_hbm.at[pl.ds(base + j * _CHUNK, _CHUNK)],
                                  wsems[b]).wait()

        start_gather(0, 0)
        start_gather(1, 1)
        start_gather(2, 2)
        wait_gather(0, 0)
        start_wb(0, 0)
        start_gather(3, 3)
        wait_gather(1, 1)
        start_wb(1, 1)
        wait_wb(0, 0)
        start_gather(4, 0)
        wait_gather(2, 2)
        start_wb(2, 2)
        wait_wb(1, 1)
        start_gather(5, 1)
        wait_gather(3, 3)
        start_wb(3, 3)

        def body(g, _):
            for kk in range(4):
                j = g * 4 + kk
                bn = (kk + 2) % 4
                wait_wb(j - 2, bn)
                start_gather(j + 2, bn)
                wait_gather(j, kk)
                start_wb(j, kk)
            return 0

        lax.fori_loop(1, n_chunks // 4 - 1, body, 0)

        g = n_chunks // 4 - 1
        for kk in range(4):
            j = g * 4 + kk
            bn = (kk + 2) % 4
            if kk < 2:
                wait_wb(j - 2, bn)
                start_gather(j + 2, bn)
            else:
                wait_wb(j - 2, bn)
            wait_gather(j, kk)
            start_wb(j, kk)
        wait_wb(g * 4 + 2, 2)
        wait_wb(g * 4 + 3, 3)

    return k(flat_table, gidx)


def _mlp_body(x_ref, xc_ref, w1a_ref, b1_ref, w2_ref, b2_ref,
              w3_ref, b3_ref, w4_ref, b4_ref, out_ref, xb_ref):
    bf = jnp.bfloat16
    for f in range(_N_FIELDS):
        xb_ref[:, f * _EMB:(f + 1) * _EMB] = x_ref[f].astype(bf)
    h = jnp.dot(xc_ref[...].astype(bf), w1a_ref[_N_FIELDS * _EMB:, :],
                preferred_element_type=jnp.float32)
    h += jnp.dot(xb_ref[...], w1a_ref[: _N_FIELDS * _EMB, :],
                 preferred_element_type=jnp.float32)
    h = jnp.maximum(h + b1_ref[...], 0.0)
    h = jnp.dot(h.astype(bf), w2_ref[...], preferred_element_type=jnp.float32)
    h = jnp.maximum(h + b2_ref[...], 0.0)
    h = jnp.dot(h.astype(bf), w3_ref[...], preferred_element_type=jnp.float32)
    h = jnp.maximum(h + b3_ref[...], 0.0)
    out_ref[...] = (jnp.dot(h.astype(bf), w4_ref[...],
                            preferred_element_type=jnp.float32) + b4_ref[...])


def _tc_mlp(cat, xcont, W1a, b1, W2, b2, W3, b3, W4, b4, batch=_BATCH,
            bt=1024):
    bt = min(bt, batch)
    n_steps = batch // bt
    full = lambda shape: pl.BlockSpec(shape, lambda i: (0,) * len(shape))
    return pl.pallas_call(
        _mlp_body,
        grid=(n_steps,),
        in_specs=[
            pl.BlockSpec((_N_FIELDS, bt, _EMB), lambda i: (0, i, 0)),
            pl.BlockSpec((bt, _N_CONT), lambda i: (i, 0)),
            full(W1a.shape),
            full(b1.shape),
            full(W2.shape),
            full(b2.shape),
            full(W3.shape),
            full(b3.shape),
            full(W4.shape),
            full(b4.shape),
        ],
        out_specs=pl.BlockSpec((bt, 1), lambda i: (i, 0)),
        out_shape=jax.ShapeDtypeStruct((batch, 1), jnp.float32),
        scratch_shapes=[pltpu.VMEM((bt, _N_FIELDS * _EMB), jnp.bfloat16)],
        compiler_params=pltpu.CompilerParams(
            dimension_semantics=("arbitrary",),
        ),
    )(cat, xcont, W1a, b1, W2, b2, W3, b3, W4, b4)


_SPLITS = (5120, 4096, 3072, 2048, 1024, 1024)


def kernel(xcat_batch, xcont_batch, emb_tables, W1, b1, W2, b2, W3, b3, W4, b4):
    flat_table = emb_tables.reshape(_N_FIELDS * _VOCAB, _EMB)
    bf = jnp.bfloat16
    W1c = W1.astype(bf)
    W2c, W3c, W4c = W2.astype(bf), W3.astype(bf), W4.astype(bf)

    offs = (jnp.arange(_N_FIELDS, dtype=jnp.int32) * _VOCAB)[:, None]
    gidx_all = xcat_batch.T + offs

    outs = []
    c0 = 0
    for bc in _SPLITS:
        rows_per_w = _N_FIELDS * bc // _NW
        chunk = next(ch for ch in (128, 104, 64, 52, 32, 26, 16, 8)
                     if rows_per_w % ch == 0 and (rows_per_w // ch) % 4 == 0)
        n_chunks = rows_per_w // chunk
        gidx = gidx_all[:, c0:c0 + bc].reshape(_NW, n_chunks, chunk)
        cat = _sc_gather(flat_table, gidx, chunk, n_chunks)
        cat = cat.reshape(_N_FIELDS, bc, _EMB)
        xc = xcont_batch[c0:c0 + bc]
        outs.append(_tc_mlp(cat, xc, W1c, b1, W2c, b2, W3c, b3, W4c, b4,
                            batch=bc))
        c0 += bc
    return jnp.concatenate(outs, axis=0)


# speedup vs baseline: 1.0752x; 1.0752x over previous
import functools

import jax
import jax.numpy as jnp
from jax import lax
from jax.experimental import pallas as pl
from jax.experimental.pallas import tpu as pltpu
from jax.experimental.pallas import tpu_sc as plsc

_N_FIELDS = 26
_VOCAB = 1000
_EMB = 128
_N_CONT = 13
_BATCH = 16384

_NW = 32


def _sc_gather(flat_table, gidx, chunk, n_chunks):
    mesh = plsc.VectorSubcoreMesh(core_axis_name="c", subcore_axis_name="s")
    rows_per_w = n_chunks * chunk
    _CHUNK = chunk

    @functools.partial(
        pl.kernel,
        mesh=mesh,
        out_type=jax.ShapeDtypeStruct((_NW * rows_per_w, _EMB), jnp.float32),
        scratch_types=[
            pltpu.VMEM((n_chunks, chunk), jnp.int32),
            [pltpu.VMEM((chunk, _EMB), jnp.float32)] * 4,
            [pltpu.SemaphoreType.DMA] * 4,
            [pltpu.SemaphoreType.DMA] * 4,
        ],
    )
    def k(tbl_hbm, idx_hbm, out_hbm, idx_v, bufs, gsems, wsems):
        wid = lax.axis_index("s") * 2 + lax.axis_index("c")
        base = wid * rows_per_w
        pltpu.sync_copy(idx_hbm.at[wid], idx_v)

        def start_gather(j, b):
            pltpu.async_copy(tbl_hbm.at[idx_v.at[j]], bufs[b], gsems[b])

        def wait_gather(j, b):
            pltpu.make_async_copy(tbl_hbm.at[idx_v.at[j]], bufs[b],
                                  gsems[b]).wait()

        def start_wb(j, b):
            pltpu.async_copy(bufs[b],
                             out_hbm.at[pl.ds(base + j * _CHUNK, _CHUNK)],
                             wsems[b])

        def wait_wb(j, b):
            pltpu.make_async_copy(bufs[b],
                                  out_hbm.at[pl.ds(base + j * _CHUNK, _CHUNK)],
                                  wsems[b]).wait()

        start_gather(0, 0)
        start_gather(1, 1)
        start_gather(2, 2)
        wait_gather(0, 0)
        start_wb(0, 0)
        start_gather(3, 3)
        wait_gather(1, 1)
        start_wb(1, 1)
        wait_wb(0, 0)
        start_gather(4, 0)
        wait_gather(2, 2)
        start_wb(2, 2)
        wait_wb(1, 1)
        start_gather(5, 1)
        wait_gather(3, 3)
        start_wb(3, 3)

        def body(g, _):
            for kk in range(4):
                j = g * 4 + kk
                bn = (kk + 2) % 4
                wait_wb(j - 2, bn)
                start_gather(j + 2, bn)
                wait_gather(j, kk)
                start_wb(j, kk)
            return 0

        lax.fori_loop(1, n_chunks // 4 - 1, body, 0)

        g = n_chunks // 4 - 1
        for kk in range(4):
            j = g * 4 + kk
            bn = (kk + 2) % 4
            if kk < 2:
                wait_wb(j - 2, bn)
                start_gather(j + 2, bn)
            else:
                wait_wb(j - 2, bn)
            wait_gather(j, kk)
            start_wb(j, kk)
        wait_wb(g * 4 + 2, 2)
        wait_wb(g * 4 + 3, 3)

    return k(flat_table, gidx)


def _mlp_body(x_ref, xc_ref, w1a_ref, b1_ref, w2_ref, b2_ref,
              w3_ref, b3_ref, w4_ref, b4_ref, out_ref, xb_ref):
    bf = jnp.bfloat16
    for f in range(_N_FIELDS):
        xb_ref[:, f * _EMB:(f + 1) * _EMB] = x_ref[f].astype(bf)
    h = jnp.dot(xc_ref[...].astype(bf), w1a_ref[_N_FIELDS * _EMB:, :],
                preferred_element_type=jnp.float32)
    h += jnp.dot(xb_ref[...], w1a_ref[: _N_FIELDS * _EMB, :],
                 preferred_element_type=jnp.float32)
    h = jnp.maximum(h + b1_ref[...], 0.0)
    h = jnp.dot(h.astype(bf), w2_ref[...], preferred_element_type=jnp.float32)
    h = jnp.maximum(h + b2_ref[...], 0.0)
    h = jnp.dot(h.astype(bf), w3_ref[...], preferred_element_type=jnp.float32)
    h = jnp.maximum(h + b3_ref[...], 0.0)
    out_ref[...] = (jnp.dot(h.astype(bf), w4_ref[...],
                            preferred_element_type=jnp.float32) + b4_ref[...])


def _tc_mlp(cat, xcont, W1a, b1, W2, b2, W3, b3, W4, b4, batch=_BATCH,
            bt=1024):
    bt = min(bt, batch)
    n_steps = batch // bt
    full = lambda shape: pl.BlockSpec(shape, lambda i: (0,) * len(shape))
    return pl.pallas_call(
        _mlp_body,
        grid=(n_steps,),
        in_specs=[
            pl.BlockSpec((_N_FIELDS, bt, _EMB), lambda i: (0, i, 0)),
            pl.BlockSpec((bt, _N_CONT), lambda i: (i, 0)),
            full(W1a.shape),
            full(b1.shape),
            full(W2.shape),
            full(b2.shape),
            full(W3.shape),
            full(b3.shape),
            full(W4.shape),
            full(b4.shape),
        ],
        out_specs=pl.BlockSpec((bt, 1), lambda i: (i, 0)),
        out_shape=jax.ShapeDtypeStruct((batch, 1), jnp.float32),
        scratch_shapes=[pltpu.VMEM((bt, _N_FIELDS * _EMB), jnp.bfloat16)],
        compiler_params=pltpu.CompilerParams(
            dimension_semantics=("arbitrary",),
        ),
    )(cat, xcont, W1a, b1, W2, b2, W3, b3, W4, b4)


_SPLITS = (5120, 5120, 5120, 1024)


def kernel(xcat_batch, xcont_batch, emb_tables, W1, b1, W2, b2, W3, b3, W4, b4):
    flat_table = emb_tables.reshape(_N_FIELDS * _VOCAB, _EMB)
    bf = jnp.bfloat16
    W1c = W1.astype(bf)
    W2c, W3c, W4c = W2.astype(bf), W3.astype(bf), W4.astype(bf)

    offs = (jnp.arange(_N_FIELDS, dtype=jnp.int32) * _VOCAB)[:, None]
    gidx_all = xcat_batch.T + offs

    outs = []
    c0 = 0
    for bc in _SPLITS:
        rows_per_w = _N_FIELDS * bc // _NW
        chunk = next(ch for ch in (128, 104, 64, 52, 32, 26, 16, 8)
                     if rows_per_w % ch == 0 and (rows_per_w // ch) % 4 == 0)
        n_chunks = rows_per_w // chunk
        gidx = gidx_all[:, c0:c0 + bc].reshape(_NW, n_chunks, chunk)
        cat = _sc_gather(flat_table, gidx, chunk, n_chunks)
        cat = cat.reshape(_N_FIELDS, bc, _EMB)
        xc = xcont_batch[c0:c0 + bc]
        outs.append(_tc_mlp(cat, xc, W1c, b1, W2c, b2, W3c, b3, W4c, b4,
                            batch=bc))
        c0 += bc
    return jnp.concatenate(outs, axis=0)


# speedup vs baseline: 1.0930x; 1.0165x over previous
import functools

import jax
import jax.numpy as jnp
from jax import lax
from jax.experimental import pallas as pl
from jax.experimental.pallas import tpu as pltpu
from jax.experimental.pallas import tpu_sc as plsc

_N_FIELDS = 26
_VOCAB = 1000
_EMB = 128
_N_CONT = 13
_BATCH = 16384

_NW = 32


def _sc_gather(flat_table, gidx, chunk, n_chunks):
    mesh = plsc.VectorSubcoreMesh(core_axis_name="c", subcore_axis_name="s")
    rows_per_w = n_chunks * chunk
    _CHUNK = chunk

    @functools.partial(
        pl.kernel,
        mesh=mesh,
        out_type=jax.ShapeDtypeStruct((_NW * rows_per_w, _EMB), jnp.float32),
        scratch_types=[
            pltpu.VMEM((n_chunks, chunk), jnp.int32),
            [pltpu.VMEM((chunk, _EMB), jnp.float32)] * 4,
            [pltpu.SemaphoreType.DMA] * 4,
            [pltpu.SemaphoreType.DMA] * 4,
        ],
    )
    def k(tbl_hbm, idx_hbm, out_hbm, idx_v, bufs, gsems, wsems):
        wid = lax.axis_index("s") * 2 + lax.axis_index("c")
        base = wid * rows_per_w
        pltpu.sync_copy(idx_hbm.at[wid], idx_v)

        def start_gather(j, b):
            pltpu.async_copy(tbl_hbm.at[idx_v.at[j]], bufs[b], gsems[b])

        def wait_gather(j, b):
            pltpu.make_async_copy(tbl_hbm.at[idx_v.at[j]], bufs[b],
                                  gsems[b]).wait()

        def start_wb(j, b):
            pltpu.async_copy(bufs[b],
                             out_hbm.at[pl.ds(base + j * _CHUNK, _CHUNK)],
                             wsems[b])

        def wait_wb(j, b):
            pltpu.make_async_copy(bufs[b],
                                  out_hbm.at[pl.ds(base + j * _CHUNK, _CHUNK)],
                                  wsems[b]).wait()

        start_gather(0, 0)
        start_gather(1, 1)
        start_gather(2, 2)
        wait_gather(0, 0)
        start_wb(0, 0)
        start_gather(3, 3)
        wait_gather(1, 1)
        start_wb(1, 1)
        wait_wb(0, 0)
        start_gather(4, 0)
        wait_gather(2, 2)
        start_wb(2, 2)
        wait_wb(1, 1)
        start_gather(5, 1)
        wait_gather(3, 3)
        start_wb(3, 3)

        def body(g, _):
            for kk in range(4):
                j = g * 4 + kk
                bn = (kk + 2) % 4
                wait_wb(j - 2, bn)
                start_gather(j + 2, bn)
                wait_gather(j, kk)
                start_wb(j, kk)
            return 0

        lax.fori_loop(1, n_chunks // 4 - 1, body, 0)

        g = n_chunks // 4 - 1
        for kk in range(4):
            j = g * 4 + kk
            bn = (kk + 2) % 4
            if kk < 2:
                wait_wb(j - 2, bn)
                start_gather(j + 2, bn)
            else:
                wait_wb(j - 2, bn)
            wait_gather(j, kk)
            start_wb(j, kk)
        wait_wb(g * 4 + 2, 2)
        wait_wb(g * 4 + 3, 3)

    return k(flat_table, gidx)


def _mlp_body(x_ref, xc_ref, w1a_ref, b1_ref, w2_ref, b2_ref,
              w3_ref, b3_ref, w4_ref, b4_ref, out_ref, xb_ref):
    bf = jnp.bfloat16
    for f in range(_N_FIELDS):
        xb_ref[:, f * _EMB:(f + 1) * _EMB] = x_ref[f].astype(bf)
    h = jnp.dot(xc_ref[...].astype(bf), w1a_ref[_N_FIELDS * _EMB:, :],
                preferred_element_type=jnp.float32)
    h += jnp.dot(xb_ref[...], w1a_ref[: _N_FIELDS * _EMB, :],
                 preferred_element_type=jnp.float32)
    h = jnp.maximum(h + b1_ref[...], 0.0)
    h = jnp.dot(h.astype(bf), w2_ref[...], preferred_element_type=jnp.float32)
    h = jnp.maximum(h + b2_ref[...], 0.0)
    h = jnp.dot(h.astype(bf), w3_ref[...], preferred_element_type=jnp.float32)
    h = jnp.maximum(h + b3_ref[...], 0.0)
    out_ref[...] = (jnp.dot(h.astype(bf), w4_ref[...],
                            preferred_element_type=jnp.float32) + b4_ref[...])


def _tc_mlp(cat, xcont, W1a, b1, W2, b2, W3, b3, W4, b4, batch=_BATCH,
            bt=1024):
    bt = min(bt, batch)
    n_steps = batch // bt
    full = lambda shape: pl.BlockSpec(shape, lambda i: (0,) * len(shape))
    return pl.pallas_call(
        _mlp_body,
        grid=(n_steps,),
        in_specs=[
            pl.BlockSpec((_N_FIELDS, bt, _EMB), lambda i: (0, i, 0)),
            pl.BlockSpec((bt, _N_CONT), lambda i: (i, 0)),
            full(W1a.shape),
            full(b1.shape),
            full(W2.shape),
            full(b2.shape),
            full(W3.shape),
            full(b3.shape),
            full(W4.shape),
            full(b4.shape),
        ],
        out_specs=pl.BlockSpec((bt, 1), lambda i: (i, 0)),
        out_shape=jax.ShapeDtypeStruct((batch, 1), jnp.float32),
        scratch_shapes=[pltpu.VMEM((bt, _N_FIELDS * _EMB), jnp.bfloat16)],
        compiler_params=pltpu.CompilerParams(
            dimension_semantics=("arbitrary",),
        ),
    )(cat, xcont, W1a, b1, W2, b2, W3, b3, W4, b4)


_SPLITS = (4096, 5120, 5120, 2048)


def kernel(xcat_batch, xcont_batch, emb_tables, W1, b1, W2, b2, W3, b3, W4, b4):
    flat_table = emb_tables.reshape(_N_FIELDS * _VOCAB, _EMB)
    bf = jnp.bfloat16
    W1c = W1.astype(bf)
    W2c, W3c, W4c = W2.astype(bf), W3.astype(bf), W4.astype(bf)

    offs = (jnp.arange(_N_FIELDS, dtype=jnp.int32) * _VOCAB)[:, None]
    gidx_all = xcat_batch.T + offs

    outs = []
    c0 = 0
    for bc in _SPLITS:
        rows_per_w = _N_FIELDS * bc // _NW
        chunk = next(ch for ch in (128, 104, 64, 52, 32, 26, 16, 8)
                     if rows_per_w % ch == 0 and (rows_per_w // ch) % 4 == 0)
        n_chunks = rows_per_w // chunk
        gidx = gidx_all[:, c0:c0 + bc].reshape(_NW, n_chunks, chunk)
        cat = _sc_gather(flat_table, gidx, chunk, n_chunks)
        cat = cat.reshape(_N_FIELDS, bc, _EMB)
        xc = xcont_batch[c0:c0 + bc]
        outs.append(_tc_mlp(cat, xc, W1c, b1, W2c, b2, W3c, b3, W4c, b4,
                            batch=bc))
        c0 += bc
    return jnp.concatenate(outs, axis=0)
